# Initial kernel scaffold; baseline (speedup 1.0000x reference)
import jax
import jax.numpy as jnp
from jax.experimental import pallas as pl


def kernel(x):
    raise NotImplementedError("write your pallas kernel here")


# speedup vs baseline: 1287.7322x; 1287.7322x over previous
import jax
import jax.numpy as jnp
from jax.experimental import pallas as pl
from jax.experimental.pallas import tpu as pltpu

_NBINS = 32
_ROWS_PER_BLOCK = 512
_FEATS = 128


def _hist_body(b_ref, x_ref, o_ref):
    x = x_ref[...]
    ge = []
    for i in range(_NBINS - 1):
        m = (x >= b_ref[i]).astype(jnp.float32)
        ge.append(jnp.sum(m, axis=1, keepdims=True))
    cols = [jnp.full((_ROWS_PER_BLOCK, 1), 128.0, jnp.float32) - ge[0]]
    for j in range(1, _NBINS - 1):
        cols.append(ge[j - 1] - ge[j])
    cols.append(ge[_NBINS - 2])
    counts = jnp.concatenate(cols, axis=1)
    o_ref[...] = counts * jnp.float32(1.0 / 128.0)


def kernel(x):
    B, T, F = x.shape
    rows = B * T
    x2 = x.reshape(rows, F)
    boundaries = jnp.linspace(-3.0, 3.0, _NBINS - 1).astype(jnp.float32)
    grid = (rows // _ROWS_PER_BLOCK,)
    out = pl.pallas_call(
        _hist_body,
        grid=grid,
        in_specs=[
            pl.BlockSpec(memory_space=pltpu.SMEM),
            pl.BlockSpec((_ROWS_PER_BLOCK, F), lambda i: (i, 0)),
        ],
        out_specs=pl.BlockSpec((_ROWS_PER_BLOCK, _NBINS), lambda i: (i, 0)),
        out_shape=jax.ShapeDtypeStruct((rows, _NBINS), jnp.float32),
    )(boundaries, x2)
    return out.reshape(B, T, _NBINS)
